# baseline (speedup 1.0000x reference)
import functools

import jax
import jax.numpy as jnp
from jax import lax
from jax.experimental import pallas as pl
from jax.experimental.pallas import tpu as pltpu
from jax.experimental.pallas import tpu_sc as plsc

N = 10000
E = 320000
H = 128
L = 4

NC = 2
NS = 16
NW = NC * NS

CHUNK = 56
CPS = 180
EPS = CHUNK * CPS
E_PAD = EPS * NW
N_PAD = 10112
ROWS_PER_TILE = N_PAD // NS

EC_BLK = 2048


def _ec_body(ef_ref, ew1_ref, eb1_ref, ew2_ref, eb2_ref, we_ref, mb1_ref,
             o0_ref, o1_ref, o2_ref, o3_ref):
    ef = ef_ref[...]
    t = jnp.maximum(ef @ ew1_ref[...] + eb1_ref[...], 0.0)
    e = t @ ew2_ref[...] + eb2_ref[...]
    outs = (o0_ref, o1_ref, o2_ref, o3_ref)
    for l in range(L):
        outs[l][...] = e @ we_ref[l] + mb1_ref[l]


def _compute_ec(ef_pad, ew1, eb1, ew2, eb2, we_stack, mb1_stack):
    n_blk = E_PAD // EC_BLK
    full = lambda shape: pl.BlockSpec(shape, lambda i: (0,) * len(shape))
    return pl.pallas_call(
        _ec_body,
        grid=(n_blk,),
        in_specs=[
            pl.BlockSpec((EC_BLK, 2), lambda i: (i, 0)),
            full((2, H)), full((1, H)), full((H, H)), full((1, H)),
            full((L, H, H)), full((L, 1, H)),
        ],
        out_specs=[pl.BlockSpec((EC_BLK, H), lambda i: (i, 0))] * L,
        out_shape=[jax.ShapeDtypeStruct((E_PAD, H), jnp.float32)] * L,
    )(ef_pad, ew1, eb1, ew2, eb2, we_stack, mb1_stack)


def _node_enc_body(nf_ref, w1_ref, b1_ref, w2_ref, b2_ref, ws_ref, wd_ref,
                   h_ref, a_ref, b_ref):
    t = jnp.maximum(nf_ref[...] @ w1_ref[...] + b1_ref[...], 0.0)
    h = t @ w2_ref[...] + b2_ref[...]
    h_ref[...] = h
    a_ref[...] = h @ ws_ref[...]
    b_ref[...] = h @ wd_ref[...]


def _node_encode(nf, w1, b1, w2, b2, ws, wd):
    return pl.pallas_call(
        _node_enc_body,
        out_shape=[jax.ShapeDtypeStruct((N, H), jnp.float32)] * 3,
    )(nf, w1, b1, w2, b2, ws, wd)


def _update_body(s_ref, deg_ref, h_ref, mw2_ref, mb2_ref, u1a_ref, u1b_ref,
                 ub1_ref, uw2_ref, ub2_ref, wsn_ref, wdn_ref,
                 h_ref_o, a_ref_o, b_ref_o):
    s = s_ref[0, :N, :] + s_ref[1, :N, :]
    deg = deg_ref[0, :N, 0:1] + deg_ref[1, :N, 0:1]
    h = h_ref[...]
    h_agg = s @ mw2_ref[...] + deg * mb2_ref[...]
    u = jnp.maximum(h @ u1a_ref[...] + h_agg @ u1b_ref[...] + ub1_ref[...],
                    0.0) @ uw2_ref[...] + ub2_ref[...]
    hn = u + h
    h_ref_o[...] = hn
    a_ref_o[...] = hn @ wsn_ref[...]
    b_ref_o[...] = hn @ wdn_ref[...]


def _update(s, deg16, h, mw2, mb2, u1a, u1b, ub1, uw2, ub2, wsn, wdn):
    return pl.pallas_call(
        _update_body,
        out_shape=[jax.ShapeDtypeStruct((N, H), jnp.float32)] * 3,
    )(s, deg16, h, mw2, mb2, u1a, u1b, ub1, uw2, ub2, wsn, wdn)


def _final_body(h_ref, gw1_ref, gb1_ref, gw2_ref, gb2_ref, hw_ref, hb_ref,
                ge_ref, ho_ref):
    h = h_ref[...]
    g = jnp.maximum(h @ gw1_ref[...] + gb1_ref[...], 0.0) @ gw2_ref[...] \
        + gb2_ref[...]
    g = g - jnp.max(g)
    a = jnp.exp(g)
    alpha = a / jnp.sum(a)
    ge = jnp.sum(alpha * h, axis=0, keepdims=True)
    ge_ref[...] = ge
    ho_ref[...] = ge @ hw_ref[...] + hb_ref[...]


def _final(h, gw1, gb1, gw2, gb2, hw_t, hb_t):
    return pl.pallas_call(
        _final_body,
        out_shape=[jax.ShapeDtypeStruct((1, H), jnp.float32),
                   jax.ShapeDtypeStruct((1, L), jnp.float32)],
    )(h, gw1, gb1, gw2, gb2, hw_t, hb_t)


_SC_MESH = plsc.VectorSubcoreMesh(core_axis_name="c", subcore_axis_name="s",
                                  num_cores=NC, num_subcores=NS)


@functools.partial(
    pl.kernel,
    mesh=_SC_MESH,
    out_type=jax.ShapeDtypeStruct((NC, N_PAD, H), jnp.float32),
    scratch_types=[
        [pltpu.VMEM((2, CHUNK), jnp.int32)] * 4,
        [pltpu.VMEM((CHUNK, H), jnp.float32)] * 2,
        [pltpu.VMEM((CHUNK, H), jnp.float32)] * 2,
        [pltpu.VMEM((CHUNK, H), jnp.float32)] * 2,
        pltpu.VMEM_SHARED((N_PAD, H), jnp.float32),
        [pltpu.SemaphoreType.DMA] * 4,
        [pltpu.SemaphoreType.DMA] * 2,
        [pltpu.SemaphoreType.DMA] * 2,
        [pltpu.SemaphoreType.DMA] * 2,
    ],
)
def _sc_edge_pass(a_hbm, b_hbm, ec_hbm, sd_hbm, zeros_hbm, out_hbm,
                  sds, avs, bvs, ecs, s_sh,
                  sis, sas, sbs, ses):
    c = lax.axis_index("c")
    s = lax.axis_index("s")
    wid = c * NS + s
    tile_base = s * ROWS_PER_TILE

    pltpu.sync_copy(zeros_hbm, s_sh.at[pl.ds(tile_base, ROWS_PER_TILE)])
    plsc.subcore_barrier()

    def idx_load(k, j):
        pltpu.async_copy(sd_hbm.at[wid * CPS + k], sds[j], sis[j])

    def idx_wait(j):
        pltpu.make_async_copy(sd_hbm.at[0], sds[j], sis[j]).wait()

    def gathers_start(k, j, b):
        ebase = wid * EPS + k * CHUNK
        pltpu.async_copy(a_hbm.at[sds[j].at[0]], avs[b], sas[b])
        pltpu.async_copy(b_hbm.at[sds[j].at[1]], bvs[b], sbs[b])
        pltpu.async_copy(ec_hbm.at[pl.ds(ebase, CHUNK)], ecs[b], ses[b])

    def gathers_wait(j, b):
        pltpu.make_async_copy(a_hbm.at[sds[j].at[0]], avs[b], sas[b]).wait()
        pltpu.make_async_copy(b_hbm.at[sds[j].at[1]], bvs[b], sbs[b]).wait()
        pltpu.make_async_copy(ec_hbm.at[pl.ds(0, CHUNK)], ecs[b],
                              ses[b]).wait()

    for j in range(4):
        idx_load(j, j)
    for b in range(2):
        idx_wait(b)
        gathers_start(b, b, b)

    def pipe_body(qq, carry):
        for jj in range(4):
            k = 4 * qq + jj
            b = jj % 2
            j = jj

            @pl.when(k < CPS)
            def _():
                gathers_wait(j, b)
                a_v, b_v, ec_v = avs[b], bvs[b], ecs[b]

                @plsc.parallel_loop(0, CHUNK, step=1, unroll=4)
                def row_body(r):
                    for i in range(H // 16):
                        sl = pl.ds(16 * i, 16)
                        ec_v[r, sl] = jnp.maximum(
                            a_v[r, sl] + b_v[r, sl] + ec_v[r, sl], 0.0)

                pltpu.sync_copy(ec_v, s_sh.at[sds[j].at[1]], add=True)

                @pl.when(k + 2 < CPS)
                def _():
                    j2 = (j + 2) % 4
                    idx_wait(j2)
                    gathers_start(k + 2, j2, b)

                @pl.when(k + 4 < CPS)
                def _():
                    idx_load(k + 4, j)

        return carry

    lax.fori_loop(0, (CPS + 3) // 4, pipe_body, 0, unroll=False)

    plsc.subcore_barrier()
    pltpu.sync_copy(s_sh.at[pl.ds(tile_base, ROWS_PER_TILE)],
                    out_hbm.at[c, pl.ds(tile_base, ROWS_PER_TILE)])


@functools.partial(
    pl.kernel,
    mesh=_SC_MESH,
    out_type=jax.ShapeDtypeStruct((NC, N_PAD, 16), jnp.float32),
    scratch_types=[
        pltpu.VMEM((CHUNK,), jnp.int32),
        pltpu.VMEM((CHUNK, 16), jnp.float32),
        pltpu.VMEM_SHARED((N_PAD, 16), jnp.float32),
    ],
)
def _sc_degree(dst_hbm, zeros_hbm, out_hbm, dst_v, ones_v, s_sh):
    c = lax.axis_index("c")
    s = lax.axis_index("s")
    wid = c * NS + s
    tile_base = s * ROWS_PER_TILE

    pltpu.sync_copy(zeros_hbm, s_sh.at[pl.ds(tile_base, ROWS_PER_TILE)])

    pattern = jnp.where(lax.iota(jnp.int32, 16) == 0, 1.0, 0.0)

    def fill_body(r, carry):
        ones_v[r, :] = pattern
        return carry

    lax.fori_loop(0, CHUNK, fill_body, 0, unroll=False)
    plsc.subcore_barrier()

    def chunk_body(k, carry):
        ebase = wid * EPS + k * CHUNK
        pltpu.sync_copy(dst_hbm.at[pl.ds(ebase, CHUNK)], dst_v)
        pltpu.sync_copy(ones_v, s_sh.at[dst_v], add=True)
        return carry

    lax.fori_loop(0, CPS, chunk_body, 0, unroll=False)

    plsc.subcore_barrier()
    pltpu.sync_copy(s_sh.at[pl.ds(tile_base, ROWS_PER_TILE)],
                    out_hbm.at[c, pl.ds(tile_base, ROWS_PER_TILE)])


def kernel(node_feat, edge_feat, edge_index, ne_w1, ne_b1, ne_w2, ne_b2,
           ee_w1, ee_b1, ee_w2, ee_b2, msg_w1, msg_b1, msg_w2, msg_b2,
           upd_w1, upd_b1, upd_w2, upd_b2, gate_w1, gate_b1, gate_w2, gate_b2,
           head_w, head_b):
    f32 = jnp.float32
    src_pad = jnp.pad(edge_index[0], (0, E_PAD - E))
    dst_pad = jnp.pad(edge_index[1], (0, E_PAD - E),
                      constant_values=N)
    ef_pad = jnp.pad(edge_feat, ((0, E_PAD - E), (0, 0)))
    sd = jnp.stack([src_pad.reshape(-1, CHUNK), dst_pad.reshape(-1, CHUNK)],
                   axis=1)

    we_stack = msg_w1[:, 2 * H:, :]
    mb1_stack = msg_b1[:, None, :]
    ec_list = _compute_ec(ef_pad, ee_w1, ee_b1[None, :], ee_w2,
                          ee_b2[None, :], we_stack, mb1_stack)

    h, a_n, b_n = _node_encode(node_feat, ne_w1, ne_b1[None, :], ne_w2,
                               ne_b2[None, :], msg_w1[0, :H, :],
                               msg_w1[0, H:2 * H, :])

    zeros_h = jnp.zeros((ROWS_PER_TILE, H), f32)
    zeros_16 = jnp.zeros((ROWS_PER_TILE, 16), f32)
    deg16 = _sc_degree(dst_pad, zeros_16)

    for l in range(L):
        s_part = _sc_edge_pass(a_n, b_n, ec_list[l], sd, zeros_h)
        ln = (l + 1) % L
        h, a_n, b_n = _update(
            s_part, deg16, h, msg_w2[l], msg_b2[l][None, :],
            upd_w1[l, :H, :], upd_w1[l, H:, :], upd_b1[l][None, :],
            upd_w2[l], upd_b2[l][None, :],
            msg_w1[ln, :H, :], msg_w1[ln, H:2 * H, :])

    hw_t = head_w[:, :, 0].T
    hb_t = head_b[:, 0][None, :]
    graph_emb, ho = _final(h, gate_w1, gate_b1[None, :], gate_w2,
                           gate_b2[None, :], hw_t, hb_t)
    head_out = ho.reshape(L, 1, 1)
    return (h, graph_emb, head_out)


# speedup vs baseline: 1.0374x; 1.0374x over previous
import functools

import jax
import jax.numpy as jnp
from jax import lax
from jax.experimental import pallas as pl
from jax.experimental.pallas import tpu as pltpu
from jax.experimental.pallas import tpu_sc as plsc

N = 10000
E = 320000
H = 128
L = 4

NC = 2
NS = 16
NW = NC * NS

CHUNK = 56
CPS = 180
EPS = CHUNK * CPS
E_PAD = EPS * NW
N_PAD = 10112
ROWS_PER_TILE = N_PAD // NS

EC_BLK = 2048


def _ec_body(ef_ref, ew1_ref, eb1_ref, ew2_ref, eb2_ref, we_ref, mb1_ref,
             o0_ref, o1_ref, o2_ref, o3_ref):
    ef = ef_ref[...]
    t = jnp.maximum(ef @ ew1_ref[...] + eb1_ref[...], 0.0)
    e = t @ ew2_ref[...] + eb2_ref[...]
    e16 = e.astype(jnp.bfloat16)
    outs = (o0_ref, o1_ref, o2_ref, o3_ref)
    for l in range(L):
        outs[l][...] = jnp.dot(e16, we_ref[l],
                               preferred_element_type=jnp.float32) \
            + mb1_ref[l]


def _compute_ec(ef_pad, ew1, eb1, ew2, eb2, we_stack, mb1_stack):
    n_blk = E_PAD // EC_BLK
    full = lambda shape: pl.BlockSpec(shape, lambda i: (0,) * len(shape))
    return pl.pallas_call(
        _ec_body,
        grid=(n_blk,),
        in_specs=[
            pl.BlockSpec((EC_BLK, 2), lambda i: (i, 0)),
            full((2, H)), full((1, H)), full((H, H)), full((1, H)),
            full((L, H, H)), full((L, 1, H)),
        ],
        out_specs=[pl.BlockSpec((EC_BLK, H), lambda i: (i, 0))] * L,
        out_shape=[jax.ShapeDtypeStruct((E_PAD, H), jnp.float32)] * L,
    )(ef_pad, ew1, eb1, ew2, eb2, we_stack, mb1_stack)


def _node_enc_body(nf_ref, w1_ref, b1_ref, w2_ref, b2_ref, ws_ref, wd_ref,
                   h_ref, a_ref, b_ref):
    t = jnp.maximum(nf_ref[...] @ w1_ref[...] + b1_ref[...], 0.0)
    h = t @ w2_ref[...] + b2_ref[...]
    h_ref[...] = h
    a_ref[...] = h @ ws_ref[...]
    b_ref[...] = h @ wd_ref[...]


def _node_encode(nf, w1, b1, w2, b2, ws, wd):
    return pl.pallas_call(
        _node_enc_body,
        out_shape=[jax.ShapeDtypeStruct((N, H), jnp.float32)] * 3,
    )(nf, w1, b1, w2, b2, ws, wd)


def _update_body(s_ref, deg_ref, h_ref, mw2_ref, mb2_ref, u1a_ref, u1b_ref,
                 ub1_ref, uw2_ref, ub2_ref, wsn_ref, wdn_ref,
                 h_ref_o, a_ref_o, b_ref_o):
    s = s_ref[0, :N, :] + s_ref[1, :N, :]
    deg = deg_ref[0, :N, 0:1] + deg_ref[1, :N, 0:1]
    h = h_ref[...]
    h_agg = s @ mw2_ref[...] + deg * mb2_ref[...]
    u = jnp.maximum(h @ u1a_ref[...] + h_agg @ u1b_ref[...] + ub1_ref[...],
                    0.0) @ uw2_ref[...] + ub2_ref[...]
    hn = u + h
    h_ref_o[...] = hn
    a_ref_o[...] = hn @ wsn_ref[...]
    b_ref_o[...] = hn @ wdn_ref[...]


def _update(s, deg16, h, mw2, mb2, u1a, u1b, ub1, uw2, ub2, wsn, wdn):
    return pl.pallas_call(
        _update_body,
        out_shape=[jax.ShapeDtypeStruct((N, H), jnp.float32)] * 3,
    )(s, deg16, h, mw2, mb2, u1a, u1b, ub1, uw2, ub2, wsn, wdn)


def _final_body(h_ref, gw1_ref, gb1_ref, gw2_ref, gb2_ref, hw_ref, hb_ref,
                ge_ref, ho_ref):
    h = h_ref[...]
    g = jnp.maximum(h @ gw1_ref[...] + gb1_ref[...], 0.0) @ gw2_ref[...] \
        + gb2_ref[...]
    g = g - jnp.max(g)
    a = jnp.exp(g)
    alpha = a / jnp.sum(a)
    ge = jnp.sum(alpha * h, axis=0, keepdims=True)
    ge_ref[...] = ge
    ho_ref[...] = ge @ hw_ref[...] + hb_ref[...]


def _final(h, gw1, gb1, gw2, gb2, hw_t, hb_t):
    return pl.pallas_call(
        _final_body,
        out_shape=[jax.ShapeDtypeStruct((1, H), jnp.float32),
                   jax.ShapeDtypeStruct((1, L), jnp.float32)],
    )(h, gw1, gb1, gw2, gb2, hw_t, hb_t)


_SC_MESH = plsc.VectorSubcoreMesh(core_axis_name="c", subcore_axis_name="s",
                                  num_cores=NC, num_subcores=NS)


@functools.partial(
    pl.kernel,
    mesh=_SC_MESH,
    out_type=jax.ShapeDtypeStruct((NC, N_PAD, H), jnp.float32),
    scratch_types=[
        [pltpu.VMEM((2, CHUNK), jnp.int32)] * 4,
        [pltpu.VMEM((CHUNK, H), jnp.float32)] * 2,
        [pltpu.VMEM((CHUNK, H), jnp.float32)] * 2,
        [pltpu.VMEM((CHUNK, H), jnp.float32)] * 2,
        pltpu.VMEM_SHARED((N_PAD, H), jnp.float32),
        [pltpu.SemaphoreType.DMA] * 4,
        [pltpu.SemaphoreType.DMA] * 2,
        [pltpu.SemaphoreType.DMA] * 2,
        [pltpu.SemaphoreType.DMA] * 2,
    ],
)
def _sc_edge_pass(a_hbm, b_hbm, ec_hbm, sd_hbm, zeros_hbm, out_hbm,
                  sds, avs, bvs, ecs, s_sh,
                  sis, sas, sbs, ses):
    c = lax.axis_index("c")
    s = lax.axis_index("s")
    wid = c * NS + s
    tile_base = s * ROWS_PER_TILE

    pltpu.sync_copy(zeros_hbm, s_sh.at[pl.ds(tile_base, ROWS_PER_TILE)])
    plsc.subcore_barrier()

    def idx_load(k, j):
        pltpu.async_copy(sd_hbm.at[wid * CPS + k], sds[j], sis[j])

    def idx_wait(j):
        pltpu.make_async_copy(sd_hbm.at[0], sds[j], sis[j]).wait()

    def gathers_start(k, j, b):
        ebase = wid * EPS + k * CHUNK
        pltpu.async_copy(a_hbm.at[sds[j].at[0]], avs[b], sas[b])
        pltpu.async_copy(b_hbm.at[sds[j].at[1]], bvs[b], sbs[b])
        pltpu.async_copy(ec_hbm.at[pl.ds(ebase, CHUNK)], ecs[b], ses[b])

    def gathers_wait(j, b):
        pltpu.make_async_copy(a_hbm.at[sds[j].at[0]], avs[b], sas[b]).wait()
        pltpu.make_async_copy(b_hbm.at[sds[j].at[1]], bvs[b], sbs[b]).wait()
        pltpu.make_async_copy(ec_hbm.at[pl.ds(0, CHUNK)], ecs[b],
                              ses[b]).wait()

    for j in range(4):
        idx_load(j, j)
    for b in range(2):
        idx_wait(b)
        gathers_start(b, b, b)

    def pipe_body(qq, carry):
        for jj in range(4):
            k = 4 * qq + jj
            b = jj % 2
            j = jj

            @pl.when(k < CPS)
            def _():
                gathers_wait(j, b)
                a_v, b_v, ec_v = avs[b], bvs[b], ecs[b]

                @plsc.parallel_loop(0, CHUNK, step=1, unroll=4)
                def row_body(r):
                    for i in range(H // 16):
                        sl = pl.ds(16 * i, 16)
                        ec_v[r, sl] = jnp.maximum(
                            a_v[r, sl] + b_v[r, sl] + ec_v[r, sl], 0.0)

                pltpu.sync_copy(ec_v, s_sh.at[sds[j].at[1]], add=True)

                @pl.when(k + 2 < CPS)
                def _():
                    j2 = (j + 2) % 4
                    idx_wait(j2)
                    gathers_start(k + 2, j2, b)

                @pl.when(k + 4 < CPS)
                def _():
                    idx_load(k + 4, j)

        return carry

    lax.fori_loop(0, (CPS + 3) // 4, pipe_body, 0, unroll=False)

    plsc.subcore_barrier()
    pltpu.sync_copy(s_sh.at[pl.ds(tile_base, ROWS_PER_TILE)],
                    out_hbm.at[c, pl.ds(tile_base, ROWS_PER_TILE)])


@functools.partial(
    pl.kernel,
    mesh=_SC_MESH,
    out_type=jax.ShapeDtypeStruct((NC, N_PAD, 16), jnp.float32),
    scratch_types=[
        pltpu.VMEM((CHUNK,), jnp.int32),
        pltpu.VMEM((CHUNK, 16), jnp.float32),
        pltpu.VMEM_SHARED((N_PAD, 16), jnp.float32),
    ],
)
def _sc_degree(dst_hbm, zeros_hbm, out_hbm, dst_v, ones_v, s_sh):
    c = lax.axis_index("c")
    s = lax.axis_index("s")
    wid = c * NS + s
    tile_base = s * ROWS_PER_TILE

    pltpu.sync_copy(zeros_hbm, s_sh.at[pl.ds(tile_base, ROWS_PER_TILE)])

    pattern = jnp.where(lax.iota(jnp.int32, 16) == 0, 1.0, 0.0)

    def fill_body(r, carry):
        ones_v[r, :] = pattern
        return carry

    lax.fori_loop(0, CHUNK, fill_body, 0, unroll=False)
    plsc.subcore_barrier()

    def chunk_body(k, carry):
        ebase = wid * EPS + k * CHUNK
        pltpu.sync_copy(dst_hbm.at[pl.ds(ebase, CHUNK)], dst_v)
        pltpu.sync_copy(ones_v, s_sh.at[dst_v], add=True)
        return carry

    lax.fori_loop(0, CPS, chunk_body, 0, unroll=False)

    plsc.subcore_barrier()
    pltpu.sync_copy(s_sh.at[pl.ds(tile_base, ROWS_PER_TILE)],
                    out_hbm.at[c, pl.ds(tile_base, ROWS_PER_TILE)])


def kernel(node_feat, edge_feat, edge_index, ne_w1, ne_b1, ne_w2, ne_b2,
           ee_w1, ee_b1, ee_w2, ee_b2, msg_w1, msg_b1, msg_w2, msg_b2,
           upd_w1, upd_b1, upd_w2, upd_b2, gate_w1, gate_b1, gate_w2, gate_b2,
           head_w, head_b):
    f32 = jnp.float32
    src_pad = jnp.pad(edge_index[0], (0, E_PAD - E))
    dst_pad = jnp.pad(edge_index[1], (0, E_PAD - E),
                      constant_values=N)
    ef_pad = jnp.pad(edge_feat, ((0, E_PAD - E), (0, 0)))
    sd = jnp.stack([src_pad.reshape(-1, CHUNK), dst_pad.reshape(-1, CHUNK)],
                   axis=1)

    we_stack = msg_w1[:, 2 * H:, :].astype(jnp.bfloat16)
    mb1_stack = msg_b1[:, None, :]
    ec_list = _compute_ec(ef_pad, ee_w1, ee_b1[None, :], ee_w2,
                          ee_b2[None, :], we_stack, mb1_stack)

    h, a_n, b_n = _node_encode(node_feat, ne_w1, ne_b1[None, :], ne_w2,
                               ne_b2[None, :], msg_w1[0, :H, :],
                               msg_w1[0, H:2 * H, :])

    zeros_h = jnp.zeros((ROWS_PER_TILE, H), f32)
    zeros_16 = jnp.zeros((ROWS_PER_TILE, 16), f32)
    deg16 = _sc_degree(dst_pad, zeros_16)

    for l in range(L):
        s_part = _sc_edge_pass(a_n, b_n, ec_list[l], sd, zeros_h)
        ln = (l + 1) % L
        h, a_n, b_n = _update(
            s_part, deg16, h, msg_w2[l], msg_b2[l][None, :],
            upd_w1[l, :H, :], upd_w1[l, H:, :], upd_b1[l][None, :],
            upd_w2[l], upd_b2[l][None, :],
            msg_w1[ln, :H, :], msg_w1[ln, H:2 * H, :])

    hw_t = head_w[:, :, 0].T
    hb_t = head_b[:, 0][None, :]
    graph_emb, ho = _final(h, gate_w1, gate_b1[None, :], gate_w2,
                           gate_b2[None, :], hw_t, hb_t)
    head_out = ho.reshape(L, 1, 1)
    return (h, graph_emb, head_out)
